# baseline (speedup 1.0000x reference)
import functools

import jax
import jax.numpy as jnp
from jax import lax
from jax.experimental import pallas as pl
from jax.experimental.pallas import tpu as pltpu
from jax.experimental.pallas import tpu_sc as plsc

_B = 16384
_D = 64
_NC = 2
_NS = 16
_NW = _NC * _NS
_BPW = _B // _NW
_CH = 128
_NCHUNK = _BPW // _CH
_IDX_ROWS = _B // _CH


def _body(ph, pr, pt, nh, nr, nt, ent, rel, pos_out, neg_out,
          idx_h, idx_r, idx_t, bh, br, bt, out_v, sem):
    wid = lax.axis_index("s") * _NC + lax.axis_index("c")

    def do_term(hi, ri, ti, out_hbm):
        pltpu.sync_copy(hi.at[pl.ds(wid * _NCHUNK, _NCHUNK)], idx_h)
        pltpu.sync_copy(ri.at[pl.ds(wid * _NCHUNK, _NCHUNK)], idx_r)
        pltpu.sync_copy(ti.at[pl.ds(wid * _NCHUNK, _NCHUNK)], idx_t)
        for c in range(_NCHUNK):
            cp1 = pltpu.async_copy(ent.at[idx_h.at[c]], bh, sem)
            cp2 = pltpu.async_copy(rel.at[idx_r.at[c]], br, sem)
            cp3 = pltpu.async_copy(ent.at[idx_t.at[c]], bt, sem)
            cp1.wait()
            cp2.wait()
            cp3.wait()
            for g in range(_CH // 16):
                rows = lax.iota(jnp.int32, 16) + (g * 16)

                def jbody(j, acc):
                    cols = lax.broadcast(j, (16,))
                    h = plsc.load_gather(bh, [rows, cols])
                    r = plsc.load_gather(br, [rows, cols])
                    t = plsc.load_gather(bt, [rows, cols])
                    d = h + r - t
                    return acc + d * d

                acc = lax.fori_loop(0, _D, jbody,
                                    jnp.zeros((16,), jnp.float32))
                out_v[pl.ds(c * _CH + g * 16, 16)] = acc
        pltpu.sync_copy(out_v, out_hbm.at[pl.ds(wid * _BPW, _BPW)])

    do_term(ph, pr, pt, pos_out)
    do_term(nh, nr, nt, neg_out)


@functools.partial(jax.jit)
def kernel(ph, pr, pt, nh, nr, nt, ent_embed, rel_embed):
    idxs = [x.astype(jnp.int32).reshape(_IDX_ROWS, _CH)
            for x in (ph, pr, pt, nh, nr, nt)]
    mesh = plsc.VectorSubcoreMesh(core_axis_name="c", subcore_axis_name="s",
                                  num_cores=_NC, num_subcores=_NS)
    f = pl.kernel(
        _body,
        out_type=(jax.ShapeDtypeStruct((_B,), jnp.float32),
                  jax.ShapeDtypeStruct((_B,), jnp.float32)),
        mesh=mesh,
        scratch_types=[
            pltpu.VMEM((_NCHUNK, _CH), jnp.int32),
            pltpu.VMEM((_NCHUNK, _CH), jnp.int32),
            pltpu.VMEM((_NCHUNK, _CH), jnp.int32),
            pltpu.VMEM((_CH, _D), jnp.float32),
            pltpu.VMEM((_CH, _D), jnp.float32),
            pltpu.VMEM((_CH, _D), jnp.float32),
            pltpu.VMEM((_BPW,), jnp.float32),
            pltpu.SemaphoreType.DMA,
        ],
        compiler_params=pltpu.CompilerParams(needs_layout_passes=False,
                                             use_tc_tiling_on_sc=False),
    )
    return f(*idxs, ent_embed, rel_embed)
